# baseline (speedup 1.0000x reference)
import functools

import jax
import jax.numpy as jnp
from jax import lax
from jax.experimental import pallas as pl
from jax.experimental.pallas import tpu as pltpu
from jax.experimental.pallas import tpu_sc as plsc

N = 10000
P = 10240
F = 256
FH = 128
NNZ = 160000
NSUB = 16
BK = 64
NBATCH = 160
NNZP = NSUB * NBATCH * BK
RPS = P // NSUB

_f32 = jnp.float32


def _vector_mesh():
    return plsc.VectorSubcoreMesh(core_axis_name="c", subcore_axis_name="s")


def _sc_degrees(idx4, zeros_fh, ones_fh):
    @functools.partial(
        pl.kernel,
        out_type=jax.ShapeDtypeStruct((2, P, FH), _f32),
        mesh=_vector_mesh(),
        scratch_types=[
            pltpu.VMEM((NBATCH, BK), jnp.int32),
            pltpu.VMEM((BK, FH), _f32),
            pltpu.VMEM((BK, FH), _f32),
            pltpu.VMEM_SHARED((P, FH), _f32),
        ],
    )
    def deg_kernel(idx_hbm, z_hbm, o_hbm, deg_hbm, idx_v, ones_v, z_v, acc_sh):
        cid = lax.axis_index("c")
        sid = lax.axis_index("s")

        pltpu.sync_copy(o_hbm, ones_v)

        @pl.when(sid == 0)
        def _():
            pltpu.sync_copy(z_hbm, z_v)
            for q in range(P // BK):
                pltpu.sync_copy(z_v, acc_sh.at[pl.ds(q * BK, BK)])

        plsc.subcore_barrier()

        pltpu.sync_copy(idx_hbm.at[cid, sid], idx_v)

        @pl.loop(0, NBATCH)
        def _(b):
            pltpu.sync_copy(ones_v, acc_sh.at[idx_v.at[b]], add=True)

        plsc.subcore_barrier()

        @pl.when(sid == 0)
        def _():
            pltpu.sync_copy(acc_sh, deg_hbm.at[cid])

    return deg_kernel(idx4, zeros_fh, ones_fh)


def _make_sc_segment(swap):
    s_src, s_dst = (1, 0) if swap else (0, 1)

    @functools.partial(
        pl.kernel,
        out_type=jax.ShapeDtypeStruct((2, P, FH), _f32),
        mesh=_vector_mesh(),
        scratch_types=[
            pltpu.VMEM((NBATCH // 2, BK), jnp.int32),
            pltpu.VMEM((NBATCH // 2, BK), jnp.int32),
            pltpu.VMEM((BK, FH), _f32),
            pltpu.VMEM((BK, FH), _f32),
            pltpu.VMEM_SHARED((P, FH), _f32),
            pltpu.SemaphoreType.DMA,
            pltpu.SemaphoreType.DMA,
            pltpu.SemaphoreType.DMA,
            pltpu.SemaphoreType.DMA,
        ],
    )
    def seg_kernel(idx_hbm, tab_hbm, z_hbm, out_hbm, src_v, dst_v, r0, r1,
                   acc_sh, g0, g1, s0, s1):
        cid = lax.axis_index("c")
        sid = lax.axis_index("s")

        @pl.when(sid == 0)
        def _():
            pltpu.sync_copy(z_hbm, r0)
            for q in range(P // BK):
                pltpu.sync_copy(r0, acc_sh.at[pl.ds(q * BK, BK)])

        plsc.subcore_barrier()

        tab_c = tab_hbm.at[cid]
        HB = NBATCH // 2

        for h in range(2):
            pltpu.sync_copy(idx_hbm.at[s_src, sid, pl.ds(h * HB, HB)], src_v)
            pltpu.sync_copy(idx_hbm.at[s_dst, sid, pl.ds(h * HB, HB)], dst_v)

            pltpu.async_copy(tab_c.at[src_v.at[0]], r0, g0).wait()
            pltpu.async_copy(r0, acc_sh.at[dst_v.at[0]], s0, add=True)
            pltpu.async_copy(tab_c.at[src_v.at[1]], r1, g1).wait()
            pltpu.async_copy(r1, acc_sh.at[dst_v.at[1]], s1, add=True)

            @pl.loop(1, HB // 2)
            def _(q):
                b = 2 * q
                pltpu.make_async_copy(z_hbm, r0, s0).wait()
                pltpu.async_copy(tab_c.at[src_v.at[b]], r0, g0).wait()
                pltpu.async_copy(r0, acc_sh.at[dst_v.at[b]], s0, add=True)
                pltpu.make_async_copy(z_hbm, r1, s1).wait()
                pltpu.async_copy(tab_c.at[src_v.at[b + 1]], r1, g1).wait()
                pltpu.async_copy(r1, acc_sh.at[dst_v.at[b + 1]], s1, add=True)

            pltpu.make_async_copy(z_hbm, r0, s0).wait()
            pltpu.make_async_copy(z_hbm, r1, s1).wait()

        plsc.subcore_barrier()

        @pl.when(sid == 0)
        def _():
            pltpu.sync_copy(acc_sh, out_hbm.at[cid])

    return seg_kernel


_seg_fwd = _make_sc_segment(swap=False)
_seg_bwd = _make_sc_segment(swap=True)


_RB = 1024


def _tc_stage(acc2, degs=None, deg_slot=0, bias=None, W=None, relu=False):
    grid = (P // _RB,)
    in_specs = [pl.BlockSpec((2, _RB, FH), lambda i: (0, i, 0))]
    args = [acc2]
    if degs is not None:
        in_specs.append(pl.BlockSpec((1, _RB, FH), lambda i: (deg_slot, i, 0)))
        args.append(degs)
    if bias is not None:
        in_specs.append(pl.BlockSpec((1, F), lambda i: (0, 0)))
        args.append(bias)
    if W is not None:
        in_specs.append(pl.BlockSpec((F, F), lambda i: (0, 0)))
        args.append(W)

    def body(*refs):
        it = iter(refs)
        a = next(it)
        lo = a[0]
        hi = a[1]
        if degs is not None:
            d = next(it)[0][:, 0:1]
            inv = jnp.where(d > 0.0, 1.0 / d, 0.0)
            lo = lo * inv
            hi = hi * inv
        h = jnp.concatenate([lo, hi], axis=1)
        if bias is not None:
            h = h + next(it)[...]
        if relu:
            h = jnp.maximum(h, 0.0)
        if W is not None:
            h = jnp.dot(h, next(it)[...], preferred_element_type=_f32)
        out = next(it)
        out[0] = h[:, :FH]
        out[1] = h[:, FH:]

    return pl.pallas_call(
        body,
        grid=grid,
        in_specs=in_specs,
        out_specs=pl.BlockSpec((2, _RB, FH), lambda i: (0, i, 0)),
        out_shape=jax.ShapeDtypeStruct((2, P, FH), _f32),
    )(*args)


def kernel(x, hyperedge_index, W1, b1, W2, b2):
    node = hyperedge_index[0].astype(jnp.int32)
    he = hyperedge_index[1].astype(jnp.int32)
    pad = (jnp.arange(NNZP - NNZ, dtype=jnp.int32) % (P - N)) + N
    node3 = jnp.concatenate([node, pad]).reshape(NSUB, NBATCH, BK)
    he3 = jnp.concatenate([he, pad]).reshape(NSUB, NBATCH, BK)
    idx4 = jnp.stack([node3, he3])
    x2 = jnp.concatenate([x, jnp.zeros((P - N, F), _f32)])
    x2 = jnp.stack([x2[:, :FH], x2[:, FH:]])
    b1r = b1.reshape(1, F)
    b2r = b2.reshape(1, F)
    zfh = jnp.zeros((BK, FH), _f32)

    degs = _sc_degrees(idx4, zfh, jnp.ones((BK, FH), _f32))
    xt = _tc_stage(x2, W=W1)

    a1 = _seg_fwd(idx4, xt, zfh)
    e1 = _tc_stage(a1, degs=degs, deg_slot=1)
    g1 = _seg_bwd(idx4, e1, zfh)
    y1 = _tc_stage(g1, degs=degs, deg_slot=0, bias=b1r, W=W2, relu=True)

    a2 = _seg_fwd(idx4, y1, zfh)
    e2 = _tc_stage(a2, degs=degs, deg_slot=1)
    g2 = _seg_bwd(idx4, e2, zfh)
    o = _tc_stage(g2, degs=degs, deg_slot=0, bias=b2r, relu=True)

    return jnp.concatenate([o[0, :N], o[1, :N]], axis=1)


# speedup vs baseline: 1.2298x; 1.2298x over previous
import functools

import jax
import jax.numpy as jnp
from jax import lax
from jax.experimental import pallas as pl
from jax.experimental.pallas import tpu as pltpu
from jax.experimental.pallas import tpu_sc as plsc

N = 10000
P = 10240
F = 256
FH = 128
NNZ = 160000
NSUB = 16
BK = 128
NBATCH = 80
NNZP = NSUB * NBATCH * BK
RPS = P // NSUB

_f32 = jnp.float32


def _vector_mesh():
    return plsc.VectorSubcoreMesh(core_axis_name="c", subcore_axis_name="s")


def _sc_degrees(idx4, zeros_fh, ones_fh):
    @functools.partial(
        pl.kernel,
        out_type=jax.ShapeDtypeStruct((2, P, FH), _f32),
        mesh=_vector_mesh(),
        scratch_types=[
            pltpu.VMEM((NBATCH, BK), jnp.int32),
            pltpu.VMEM((BK, FH), _f32),
            pltpu.VMEM((BK, FH), _f32),
            pltpu.VMEM_SHARED((P, FH), _f32),
        ],
    )
    def deg_kernel(idx_hbm, z_hbm, o_hbm, deg_hbm, idx_v, ones_v, z_v, acc_sh):
        cid = lax.axis_index("c")
        sid = lax.axis_index("s")

        pltpu.sync_copy(o_hbm, ones_v)

        @pl.when(sid == 0)
        def _():
            pltpu.sync_copy(z_hbm, z_v)
            for q in range(P // BK):
                pltpu.sync_copy(z_v, acc_sh.at[pl.ds(q * BK, BK)])

        plsc.subcore_barrier()

        pltpu.sync_copy(idx_hbm.at[cid, sid], idx_v)

        @pl.loop(0, NBATCH)
        def _(b):
            pltpu.sync_copy(ones_v, acc_sh.at[idx_v.at[b]], add=True)

        plsc.subcore_barrier()

        @pl.when(sid == 0)
        def _():
            pltpu.sync_copy(acc_sh, deg_hbm.at[cid])

    return deg_kernel(idx4, zeros_fh, ones_fh)


def _make_sc_segment(swap):
    s_src, s_dst = (1, 0) if swap else (0, 1)

    @functools.partial(
        pl.kernel,
        out_type=jax.ShapeDtypeStruct((2, P, FH), _f32),
        mesh=_vector_mesh(),
        scratch_types=[
            pltpu.VMEM((NBATCH // 2, BK), jnp.int32),
            pltpu.VMEM((NBATCH // 2, BK), jnp.int32),
            pltpu.VMEM((BK, FH), _f32),
            pltpu.VMEM((BK, FH), _f32),
            pltpu.VMEM_SHARED((P, FH), _f32),
            pltpu.SemaphoreType.DMA,
            pltpu.SemaphoreType.DMA,
            pltpu.SemaphoreType.DMA,
            pltpu.SemaphoreType.DMA,
        ],
    )
    def seg_kernel(idx_hbm, tab_hbm, z_hbm, out_hbm, src_v, dst_v, r0, r1,
                   acc_sh, g0, g1, s0, s1):
        cid = lax.axis_index("c")
        sid = lax.axis_index("s")

        @pl.when(sid == 0)
        def _():
            pltpu.sync_copy(z_hbm, r0)
            for q in range(P // BK):
                pltpu.sync_copy(r0, acc_sh.at[pl.ds(q * BK, BK)])

        plsc.subcore_barrier()

        tab_c = tab_hbm.at[cid]
        HB = NBATCH // 2

        for h in range(2):
            pltpu.sync_copy(idx_hbm.at[s_src, sid, pl.ds(h * HB, HB)], src_v)
            pltpu.sync_copy(idx_hbm.at[s_dst, sid, pl.ds(h * HB, HB)], dst_v)

            pltpu.async_copy(tab_c.at[src_v.at[0]], r0, g0).wait()
            pltpu.async_copy(r0, acc_sh.at[dst_v.at[0]], s0, add=True)
            pltpu.async_copy(tab_c.at[src_v.at[1]], r1, g1).wait()
            pltpu.async_copy(r1, acc_sh.at[dst_v.at[1]], s1, add=True)

            @pl.loop(1, HB // 2)
            def _(q):
                b = 2 * q
                pltpu.make_async_copy(z_hbm, r0, s0).wait()
                pltpu.async_copy(tab_c.at[src_v.at[b]], r0, g0).wait()
                pltpu.async_copy(r0, acc_sh.at[dst_v.at[b]], s0, add=True)
                pltpu.make_async_copy(z_hbm, r1, s1).wait()
                pltpu.async_copy(tab_c.at[src_v.at[b + 1]], r1, g1).wait()
                pltpu.async_copy(r1, acc_sh.at[dst_v.at[b + 1]], s1, add=True)

            pltpu.make_async_copy(z_hbm, r0, s0).wait()
            pltpu.make_async_copy(z_hbm, r1, s1).wait()

        plsc.subcore_barrier()

        @pl.when(sid == 0)
        def _():
            pltpu.sync_copy(acc_sh, out_hbm.at[cid])

    return seg_kernel


_seg_fwd = _make_sc_segment(swap=False)
_seg_bwd = _make_sc_segment(swap=True)


_RB = 1024


def _tc_stage(acc2, degs=None, deg_slot=0, bias=None, W=None, relu=False):
    grid = (P // _RB,)
    in_specs = [pl.BlockSpec((2, _RB, FH), lambda i: (0, i, 0))]
    args = [acc2]
    if degs is not None:
        in_specs.append(pl.BlockSpec((1, _RB, FH), lambda i: (deg_slot, i, 0)))
        args.append(degs)
    if bias is not None:
        in_specs.append(pl.BlockSpec((1, F), lambda i: (0, 0)))
        args.append(bias)
    if W is not None:
        in_specs.append(pl.BlockSpec((F, F), lambda i: (0, 0)))
        args.append(W)

    def body(*refs):
        it = iter(refs)
        a = next(it)
        lo = a[0]
        hi = a[1]
        if degs is not None:
            d = next(it)[0][:, 0:1]
            inv = jnp.where(d > 0.0, 1.0 / d, 0.0)
            lo = lo * inv
            hi = hi * inv
        h = jnp.concatenate([lo, hi], axis=1)
        if bias is not None:
            h = h + next(it)[...]
        if relu:
            h = jnp.maximum(h, 0.0)
        if W is not None:
            h = jnp.dot(h, next(it)[...], preferred_element_type=_f32)
        out = next(it)
        out[0] = h[:, :FH]
        out[1] = h[:, FH:]

    return pl.pallas_call(
        body,
        grid=grid,
        in_specs=in_specs,
        out_specs=pl.BlockSpec((2, _RB, FH), lambda i: (0, i, 0)),
        out_shape=jax.ShapeDtypeStruct((2, P, FH), _f32),
    )(*args)


def kernel(x, hyperedge_index, W1, b1, W2, b2):
    node = hyperedge_index[0].astype(jnp.int32)
    he = hyperedge_index[1].astype(jnp.int32)
    pad = (jnp.arange(NNZP - NNZ, dtype=jnp.int32) % (P - N)) + N
    node3 = jnp.concatenate([node, pad]).reshape(NSUB, NBATCH, BK)
    he3 = jnp.concatenate([he, pad]).reshape(NSUB, NBATCH, BK)
    idx4 = jnp.stack([node3, he3])
    x2 = jnp.concatenate([x, jnp.zeros((P - N, F), _f32)])
    x2 = jnp.stack([x2[:, :FH], x2[:, FH:]])
    b1r = b1.reshape(1, F)
    b2r = b2.reshape(1, F)
    zfh = jnp.zeros((BK, FH), _f32)

    degs = _sc_degrees(idx4, zfh, jnp.ones((BK, FH), _f32))
    xt = _tc_stage(x2, W=W1)

    a1 = _seg_fwd(idx4, xt, zfh)
    e1 = _tc_stage(a1, degs=degs, deg_slot=1)
    g1 = _seg_bwd(idx4, e1, zfh)
    y1 = _tc_stage(g1, degs=degs, deg_slot=0, bias=b1r, W=W2, relu=True)

    a2 = _seg_fwd(idx4, y1, zfh)
    e2 = _tc_stage(a2, degs=degs, deg_slot=1)
    g2 = _seg_bwd(idx4, e2, zfh)
    o = _tc_stage(g2, degs=degs, deg_slot=0, bias=b2r, relu=True)

    return jnp.concatenate([o[0, :N], o[1, :N]], axis=1)


# speedup vs baseline: 1.2335x; 1.0029x over previous
import functools

import jax
import jax.numpy as jnp
from jax import lax
from jax.experimental import pallas as pl
from jax.experimental.pallas import tpu as pltpu
from jax.experimental.pallas import tpu_sc as plsc

N = 10000
P = 10240
F = 256
FH = 128
NNZ = 160000
NSUB = 16
BK = 128
NBATCH = 80
NNZP = NSUB * NBATCH * BK
RPS = P // NSUB

_f32 = jnp.float32


def _vector_mesh():
    return plsc.VectorSubcoreMesh(core_axis_name="c", subcore_axis_name="s")


def _sc_degrees(idx4, zeros_fh, ones_fh):
    @functools.partial(
        pl.kernel,
        out_type=jax.ShapeDtypeStruct((2, P, FH), _f32),
        mesh=_vector_mesh(),
        scratch_types=[
            pltpu.VMEM((NBATCH, BK), jnp.int32),
            pltpu.VMEM((BK, FH), _f32),
            pltpu.VMEM((BK, FH), _f32),
            pltpu.VMEM_SHARED((P, FH), _f32),
            pltpu.SemaphoreType.DMA,
            pltpu.SemaphoreType.DMA,
        ],
    )
    def deg_kernel(idx_hbm, z_hbm, o_hbm, deg_hbm, idx_v, ones_v, z_v, acc_sh,
                   s0, s1):
        cid = lax.axis_index("c")
        sid = lax.axis_index("s")

        pltpu.sync_copy(o_hbm, ones_v)

        @pl.when(sid == 0)
        def _():
            pltpu.sync_copy(z_hbm, z_v)
            for q in range(P // BK):
                pltpu.sync_copy(z_v, acc_sh.at[pl.ds(q * BK, BK)])

        plsc.subcore_barrier()

        pltpu.sync_copy(idx_hbm.at[cid, sid], idx_v)

        pltpu.async_copy(ones_v, acc_sh.at[idx_v.at[0]], s0, add=True)
        pltpu.async_copy(ones_v, acc_sh.at[idx_v.at[1]], s1, add=True)

        @pl.loop(1, NBATCH // 2)
        def _(b):
            pltpu.make_async_copy(o_hbm, ones_v, s0).wait()
            pltpu.async_copy(ones_v, acc_sh.at[idx_v.at[2 * b]], s0, add=True)
            pltpu.make_async_copy(o_hbm, ones_v, s1).wait()
            pltpu.async_copy(ones_v, acc_sh.at[idx_v.at[2 * b + 1]], s1,
                             add=True)

        pltpu.make_async_copy(o_hbm, ones_v, s0).wait()
        pltpu.make_async_copy(o_hbm, ones_v, s1).wait()
        plsc.subcore_barrier()

        @pl.when(sid == 0)
        def _():
            pltpu.sync_copy(acc_sh, deg_hbm.at[cid])

    return deg_kernel(idx4, zeros_fh, ones_fh)


def _make_sc_segment(swap):
    s_src, s_dst = (1, 0) if swap else (0, 1)

    @functools.partial(
        pl.kernel,
        out_type=jax.ShapeDtypeStruct((2, P, FH), _f32),
        mesh=_vector_mesh(),
        scratch_types=[
            pltpu.VMEM((NBATCH // 2, BK), jnp.int32),
            pltpu.VMEM((NBATCH // 2, BK), jnp.int32),
            pltpu.VMEM((BK, FH), _f32),
            pltpu.VMEM((BK, FH), _f32),
            pltpu.VMEM_SHARED((P, FH), _f32),
            pltpu.SemaphoreType.DMA,
            pltpu.SemaphoreType.DMA,
            pltpu.SemaphoreType.DMA,
            pltpu.SemaphoreType.DMA,
        ],
    )
    def seg_kernel(idx_hbm, tab_hbm, z_hbm, out_hbm, src_v, dst_v, r0, r1,
                   acc_sh, g0, g1, s0, s1):
        cid = lax.axis_index("c")
        sid = lax.axis_index("s")

        @pl.when(sid == 0)
        def _():
            pltpu.sync_copy(z_hbm, r0)
            for q in range(P // BK):
                pltpu.sync_copy(r0, acc_sh.at[pl.ds(q * BK, BK)])

        plsc.subcore_barrier()

        tab_c = tab_hbm.at[cid]
        HB = NBATCH // 2

        for h in range(2):
            pltpu.sync_copy(idx_hbm.at[s_src, sid, pl.ds(h * HB, HB)], src_v)
            pltpu.sync_copy(idx_hbm.at[s_dst, sid, pl.ds(h * HB, HB)], dst_v)

            pltpu.async_copy(tab_c.at[src_v.at[0]], r0, g0).wait()
            pltpu.async_copy(r0, acc_sh.at[dst_v.at[0]], s0, add=True)
            pltpu.async_copy(tab_c.at[src_v.at[1]], r1, g1).wait()
            pltpu.async_copy(r1, acc_sh.at[dst_v.at[1]], s1, add=True)

            @pl.loop(1, HB // 2)
            def _(q):
                b = 2 * q
                pltpu.make_async_copy(z_hbm, r0, s0).wait()
                pltpu.async_copy(tab_c.at[src_v.at[b]], r0, g0).wait()
                pltpu.async_copy(r0, acc_sh.at[dst_v.at[b]], s0, add=True)
                pltpu.make_async_copy(z_hbm, r1, s1).wait()
                pltpu.async_copy(tab_c.at[src_v.at[b + 1]], r1, g1).wait()
                pltpu.async_copy(r1, acc_sh.at[dst_v.at[b + 1]], s1, add=True)

            pltpu.make_async_copy(z_hbm, r0, s0).wait()
            pltpu.make_async_copy(z_hbm, r1, s1).wait()

        plsc.subcore_barrier()

        @pl.when(sid == 0)
        def _():
            pltpu.sync_copy(acc_sh, out_hbm.at[cid])

    return seg_kernel


_seg_fwd = _make_sc_segment(swap=False)
_seg_bwd = _make_sc_segment(swap=True)


_RB = 1024


def _tc_stage(acc2, degs=None, deg_slot=0, bias=None, W=None, relu=False):
    grid = (P // _RB,)
    in_specs = [pl.BlockSpec((2, _RB, FH), lambda i: (0, i, 0))]
    args = [acc2]
    if degs is not None:
        in_specs.append(pl.BlockSpec((1, _RB, FH), lambda i: (deg_slot, i, 0)))
        args.append(degs)
    if bias is not None:
        in_specs.append(pl.BlockSpec((1, F), lambda i: (0, 0)))
        args.append(bias)
    if W is not None:
        in_specs.append(pl.BlockSpec((F, F), lambda i: (0, 0)))
        args.append(W)

    def body(*refs):
        it = iter(refs)
        a = next(it)
        lo = a[0]
        hi = a[1]
        if degs is not None:
            d = next(it)[0][:, 0:1]
            inv = jnp.where(d > 0.0, 1.0 / d, 0.0)
            lo = lo * inv
            hi = hi * inv
        h = jnp.concatenate([lo, hi], axis=1)
        if bias is not None:
            h = h + next(it)[...]
        if relu:
            h = jnp.maximum(h, 0.0)
        if W is not None:
            h = jnp.dot(h, next(it)[...], preferred_element_type=_f32)
        out = next(it)
        out[0] = h[:, :FH]
        out[1] = h[:, FH:]

    return pl.pallas_call(
        body,
        grid=grid,
        in_specs=in_specs,
        out_specs=pl.BlockSpec((2, _RB, FH), lambda i: (0, i, 0)),
        out_shape=jax.ShapeDtypeStruct((2, P, FH), _f32),
    )(*args)


def kernel(x, hyperedge_index, W1, b1, W2, b2):
    node = hyperedge_index[0].astype(jnp.int32)
    he = hyperedge_index[1].astype(jnp.int32)
    pad = (jnp.arange(NNZP - NNZ, dtype=jnp.int32) % (P - N)) + N
    node3 = jnp.concatenate([node, pad]).reshape(NSUB, NBATCH, BK)
    he3 = jnp.concatenate([he, pad]).reshape(NSUB, NBATCH, BK)
    idx4 = jnp.stack([node3, he3])
    x2 = jnp.concatenate([x, jnp.zeros((P - N, F), _f32)])
    x2 = jnp.stack([x2[:, :FH], x2[:, FH:]])
    b1r = b1.reshape(1, F)
    b2r = b2.reshape(1, F)
    zfh = jnp.zeros((BK, FH), _f32)

    degs = _sc_degrees(idx4, zfh, jnp.ones((BK, FH), _f32))
    xt = _tc_stage(x2, W=W1)

    a1 = _seg_fwd(idx4, xt, zfh)
    e1 = _tc_stage(a1, degs=degs, deg_slot=1)
    g1 = _seg_bwd(idx4, e1, zfh)
    y1 = _tc_stage(g1, degs=degs, deg_slot=0, bias=b1r, W=W2, relu=True)

    a2 = _seg_fwd(idx4, y1, zfh)
    e2 = _tc_stage(a2, degs=degs, deg_slot=1)
    g2 = _seg_bwd(idx4, e2, zfh)
    o = _tc_stage(g2, degs=degs, deg_slot=0, bias=b2r, relu=True)

    return jnp.concatenate([o[0, :N], o[1, :N]], axis=1)
